# baseline (speedup 1.0000x reference)
import functools

import jax
import jax.numpy as jnp
from jax import lax
from jax.experimental import pallas as pl
from jax.experimental.pallas import tpu as pltpu
from jax.experimental.pallas import tpu_sc as plsc

NUM_CODES = 1024
CODE_DIM = 64
BETA = 0.25

_NC = 2
_NS = 16
_L = 16


def _dist_argmin_kernel(x_ref, cb_ref, idx_ref, loss_ref, *, nb, n_elems):
    b = pl.program_id(0)
    xm = x_ref[0]
    cb = cb_ref[...]

    zc = lax.dot_general(
        cb, xm, (((1,), (0,)), ((), ())),
        preferred_element_type=jnp.float32,
        precision=lax.Precision.DEFAULT,
    )
    c2 = jnp.sum(cb * cb, axis=1, keepdims=True)
    z2 = jnp.sum(xm * xm, axis=0, keepdims=True)
    dist = z2 + c2 - 2.0 * zc

    minv = jnp.min(dist, axis=0, keepdims=True)
    iota_f = lax.broadcasted_iota(jnp.int32, dist.shape, 0).astype(jnp.float32)
    fidx = jnp.min(jnp.where(dist == minv, iota_f, 2.0 * NUM_CODES), axis=0)
    idx_ref[0, 0, :] = fidx.astype(jnp.int32)

    part = jnp.sum(minv).reshape(1, 1)
    total = jnp.where(b == 0, part, loss_ref[...] + part)
    scale = (1.0 + BETA) / n_elems
    loss_ref[...] = jnp.where(b == nb - 1, total * scale, total)


def _sc_gather_body(cbt_hbm, idx_hbm, out_hbm, cbt_v, idx_v, slab_v):
    b = lax.axis_index("s") * _NC + lax.axis_index("c")
    pltpu.sync_copy(cbt_hbm, cbt_v)
    pltpu.sync_copy(idx_hbm.at[b], idx_v)

    P = idx_v.shape[0]
    ngroups = P // _L
    half = slab_v.shape[0] // P

    for h in range(CODE_DIM // half):
        def g_body(g, carry, h=h):
            vidx = idx_v[pl.ds(g * _L, _L)]
            for dd in range(half):
                flat = vidx + (h * half + dd) * NUM_CODES
                row = plsc.load_gather(cbt_v, [flat])
                slab_v[pl.ds(dd * P + g * _L, _L)] = row
            return carry

        lax.fori_loop(0, ngroups, g_body, 0)
        pltpu.sync_copy(slab_v, out_hbm.at[b, h])


def _sc_gather(cbt_flat, idx2):
    B, P = idx2.shape
    nhalf = 2
    slab_elems = (CODE_DIM // nhalf) * P
    return pl.kernel(
        _sc_gather_body,
        out_type=jax.ShapeDtypeStruct((B, nhalf, slab_elems), jnp.float32),
        mesh=plsc.VectorSubcoreMesh(core_axis_name="c", subcore_axis_name="s"),
        compiler_params=pltpu.CompilerParams(needs_layout_passes=False),
        scratch_types=[
            pltpu.VMEM((CODE_DIM * NUM_CODES,), jnp.float32),
            pltpu.VMEM((P,), jnp.int32),
            pltpu.VMEM((slab_elems,), jnp.float32),
        ],
    )(cbt_flat, idx2)


def kernel(x, codebook):
    B, D, H, W = x.shape
    P = H * W
    x3 = x.reshape(B, D, P)

    idx2, loss = pl.pallas_call(
        functools.partial(_dist_argmin_kernel, nb=B, n_elems=x.size),
        grid=(B,),
        in_specs=[
            pl.BlockSpec((1, D, P), lambda b: (b, 0, 0)),
            pl.BlockSpec((NUM_CODES, CODE_DIM), lambda b: (0, 0)),
        ],
        out_specs=[
            pl.BlockSpec((1, 1, P), lambda b: (b, 0, 0)),
            pl.BlockSpec((1, 1), lambda b: (0, 0)),
        ],
        out_shape=[
            jax.ShapeDtypeStruct((B, 1, P), jnp.int32),
            jax.ShapeDtypeStruct((1, 1), jnp.float32),
        ],
    )(x3, codebook)
    idx2 = idx2.reshape(B, P)

    zq3 = _sc_gather(codebook.T.reshape(-1), idx2)

    z_q = zq3.reshape(B, D, H, W)
    encoding_indices = idx2.reshape(B * P)
    vq_loss = loss[0, 0]
    return (z_q, vq_loss, encoding_indices)
